# baseline (speedup 1.0000x reference)
import functools

import jax
import jax.numpy as jnp
from jax import lax
from jax.experimental import pallas as pl
from jax.experimental.pallas import tpu as pltpu
from jax.experimental.pallas import tpu_sc as plsc

_EMBED_DIM = 32
_CHUNK = 128
_INFO = plsc.get_sparse_core_info()
_NC, _NS = _INFO.num_cores, _INFO.num_subcores
_NW = _NC * _NS


def _make_gather(n_rows: int):
    assert n_rows % (_NW * _CHUNK) == 0
    rows_per_w = n_rows // _NW
    chunks_per_w = rows_per_w // _CHUNK
    mesh = plsc.VectorSubcoreMesh(core_axis_name="c", subcore_axis_name="s")

    @functools.partial(
        pl.kernel,
        mesh=mesh,
        compiler_params=pltpu.CompilerParams(use_tc_tiling_on_sc=False),
        out_type=jax.ShapeDtypeStruct((n_rows, _EMBED_DIM), jnp.bfloat16),
        scratch_types=[
            pltpu.VMEM((rows_per_w,), jnp.int32),
            pltpu.VMEM((rows_per_w, _EMBED_DIM), jnp.bfloat16),
            pltpu.SemaphoreType.DMA,
        ],
    )
    def gather(idx_hbm, table_hbm, out_hbm, idx_v, rows_v, sem):
        wid = lax.axis_index("s") * _NC + lax.axis_index("c")
        pltpu.sync_copy(idx_hbm.at[pl.ds(wid * rows_per_w, rows_per_w)],
                        idx_v)
        copies = []
        for j in range(chunks_per_w):
            copies.append(
                pltpu.async_copy(table_hbm.at[idx_v.at[pl.ds(j * _CHUNK,
                                                             _CHUNK)]],
                                 rows_v.at[pl.ds(j * _CHUNK, _CHUNK)],
                                 sem))
        for c in copies:
            c.wait()
        pltpu.sync_copy(rows_v,
                        out_hbm.at[pl.ds(wid * rows_per_w, rows_per_w)])

    return gather


def _transpose_block(src_ref, dst_ref):
    bc = src_ref.shape[1]
    p = jax.lax.broadcasted_iota(jnp.int32, (256, 256), 0)
    j = jax.lax.broadcasted_iota(jnp.int32, (256, 256), 1)
    sel = (j == 4 * (p % 64) + p // 64).astype(jnp.bfloat16)
    for g in range(bc // 256):
        a = src_ref[:, g * 256:(g + 1) * 256].astype(jnp.bfloat16)
        t = jax.lax.dot_general(sel, a, (((1,), (1,)), ((), ())),
                                preferred_element_type=jnp.float32
                                ).astype(jnp.bfloat16)
        dst_ref[g * 64:(g + 1) * 64, :] = jnp.concatenate(
            [t[m * 64:(m + 1) * 64, :] for m in range(4)], axis=1)


def _make_transpose(n_vocab: int):
    bc = 8192
    grid = (n_vocab + bc - 1) // bc
    n4 = n_vocab // 4
    return pl.pallas_call(
        _transpose_block,
        grid=(grid,),
        in_specs=[pl.BlockSpec((_EMBED_DIM, bc), lambda b: (0, b))],
        out_specs=pl.BlockSpec((bc // 4, 128), lambda b: (b, 0)),
        out_shape=jax.ShapeDtypeStruct((n4, 128), jnp.bfloat16),
    )


def kernel(x, weight):
    b, f = x.shape
    n_rows = b * f
    n_vocab, d = weight.shape
    idx = x.astype(jnp.int32).reshape(n_rows)
    w_lin = _make_transpose(n_vocab)(weight.T)
    out = _make_gather(n_rows)(idx, w_lin.reshape(n_vocab, d))
    return out.astype(jnp.float32).reshape(b, f, _EMBED_DIM)


# speedup vs baseline: 2.2867x; 2.2867x over previous
import functools

import jax
import jax.numpy as jnp
from jax import lax
from jax.experimental import pallas as pl
from jax.experimental.pallas import tpu as pltpu
from jax.experimental.pallas import tpu_sc as plsc

_EMBED_DIM = 32
_CHUNK = 128
_INFO = plsc.get_sparse_core_info()
_NC, _NS = _INFO.num_cores, _INFO.num_subcores
_NW = _NC * _NS


def _make_gather(n_rows: int):
    assert n_rows % (_NW * _CHUNK) == 0
    rows_per_w = n_rows // _NW
    chunks_per_w = rows_per_w // _CHUNK
    mesh = plsc.VectorSubcoreMesh(core_axis_name="c", subcore_axis_name="s")

    @functools.partial(
        pl.kernel,
        mesh=mesh,
        compiler_params=pltpu.CompilerParams(use_tc_tiling_on_sc=False),
        out_type=jax.ShapeDtypeStruct((n_rows, _EMBED_DIM), jnp.float32),
        scratch_types=[
            pltpu.VMEM((rows_per_w,), jnp.int32),
            pltpu.VMEM((rows_per_w, _EMBED_DIM), jnp.float32),
            pltpu.SemaphoreType.DMA,
        ],
    )
    def gather(idx_hbm, table_hbm, out_hbm, idx_v, rows_v, sem):
        wid = lax.axis_index("s") * _NC + lax.axis_index("c")
        pltpu.sync_copy(idx_hbm.at[pl.ds(wid * rows_per_w, rows_per_w)],
                        idx_v)
        copies = []
        for j in range(chunks_per_w):
            copies.append(
                pltpu.async_copy(table_hbm.at[idx_v.at[pl.ds(j * _CHUNK,
                                                             _CHUNK)]],
                                 rows_v.at[pl.ds(j * _CHUNK, _CHUNK)],
                                 sem))
        for c in copies:
            c.wait()
        pltpu.sync_copy(rows_v,
                        out_hbm.at[pl.ds(wid * rows_per_w, rows_per_w)])

    return gather


def _transpose_block(src_ref, dst_ref):
    bc = src_ref.shape[1]
    p = jax.lax.broadcasted_iota(jnp.int32, (256, 256), 0)
    j = jax.lax.broadcasted_iota(jnp.int32, (256, 256), 1)
    sel = (j == 4 * (p % 64) + p // 64).astype(jnp.float32)
    for g in range(bc // 256):
        a = src_ref[:, g * 256:(g + 1) * 256]
        t = jax.lax.dot_general(sel, a, (((1,), (1,)), ((), ())),
                                preferred_element_type=jnp.float32)
        dst_ref[g * 64:(g + 1) * 64, :] = jnp.concatenate(
            [t[m * 64:(m + 1) * 64, :] for m in range(4)], axis=1)


def _make_transpose(n_vocab: int):
    bc = 32768
    grid = (n_vocab + bc - 1) // bc
    n4 = n_vocab // 4
    return pl.pallas_call(
        _transpose_block,
        grid=(grid,),
        in_specs=[pl.BlockSpec((_EMBED_DIM, bc), lambda b: (0, b))],
        out_specs=pl.BlockSpec((bc // 4, 128), lambda b: (b, 0)),
        out_shape=jax.ShapeDtypeStruct((n4, 128), jnp.float32),
    )


def kernel(x, weight):
    b, f = x.shape
    n_rows = b * f
    n_vocab, d = weight.shape
    idx = x.astype(jnp.int32).reshape(n_rows)
    w_lin = _make_transpose(n_vocab)(weight.T)
    out = _make_gather(n_rows)(idx, w_lin.reshape(n_vocab, d))
    return out.reshape(b, f, _EMBED_DIM)
